# baseline (speedup 1.0000x reference)
import jax
import jax.numpy as jnp
from jax import lax
from jax.experimental import pallas as pl
from jax.experimental.pallas import tpu as pltpu
from jax.experimental.pallas import tpu_sc as plsc


def _pre_body(h_ref, w1t_ref, w1s_ref, b1_ref, p0_ref, p1_ref, q0_ref, q1_ref):
    h = h_ref[...]
    p = (
        jnp.dot(h, w1t_ref[...], preferred_element_type=jnp.float32)
        + b1_ref[...]
    )
    q = jnp.dot(h, w1s_ref[...], preferred_element_type=jnp.float32)
    p0_ref[...] = p[:, :128]
    p1_ref[...] = p[:, 128:]
    q0_ref[...] = q[:, :128]
    q1_ref[...] = q[:, 128:]


def _pre_call(hf, w1t, w1s, b1):
    BN, D = hf.shape
    BLK = 512
    H = D // 2
    half = jax.ShapeDtypeStruct((BN, H), jnp.float32)
    return pl.pallas_call(
        _pre_body,
        grid=(BN // BLK,),
        in_specs=[
            pl.BlockSpec((BLK, D), lambda i: (i, 0)),
            pl.BlockSpec((D, D), lambda i: (0, 0)),
            pl.BlockSpec((D, D), lambda i: (0, 0)),
            pl.BlockSpec((1, D), lambda i: (0, 0)),
        ],
        out_specs=[pl.BlockSpec((BLK, H), lambda i: (i, 0))] * 4,
        out_shape=[half, half, half, half],
    )(hf, w1t, w1s, b1)


_NC, _NS, _L = 2, 16, 16
_CH = 32
_H = 128


def _make_sc(B, N1, E):
    BN = B * N1
    rows_q = N1 // 4
    rows_w = rows_q // 2
    pad = E + 6 * _L

    def body(tgt_h, src_h, p0, p1, q0, q1, s0_out, s1_out, deg_out,
             tv, sv, stg_t, stg_s, gi0, si0, gi1, si1,
             pb0, qb0, pb1, qb1, acc, dacc,
             semp0, semq0, semp1, semq1):
        cid = lax.axis_index("c")
        sid = lax.axis_index("s")
        wid = cid * _NS + sid
        b = wid // 4
        lo = (wid % 4) * rows_q
        bbase = b * N1

        pltpu.sync_copy(tgt_h.at[b], tv.at[pl.ds(0, E)])
        pltpu.sync_copy(src_h.at[b], sv.at[pl.ds(0, E)])

        v0 = tv[pl.ds(0, _L)]
        zi = v0 - v0
        zf = zi.astype(jnp.float32)

        def _zpad(g, _):
            tv[pl.ds(E + g * _L, _L)] = zi
            sv[pl.ds(E + g * _L, _L)] = zi
            return 0
        lax.fori_loop(0, 6, _zpad, 0)

        def _zstg(g, _):
            stg_t[pl.ds(g * _L, _L)] = zi
            stg_s[pl.ds(g * _L, _L)] = zi
            return 0
        lax.fori_loop(0, pad // _L, _zstg, 0)

        def _scan(e, off):
            v = tv[pl.ds(e, _L)]
            s = sv[pl.ds(e, _L)]
            t = v[0]
            m = (t >= lo) & (t < lo + rows_q)
            stg_t[pl.ds(off, _L)] = v
            stg_s[pl.ds(off, _L)] = s
            return off + m.astype(jnp.int32)
        cnt = lax.fori_loop(0, E, _scan, 0)

        for sub in range(2):
            sublo = lo + sub * rows_w
            def _sscan(e, off):
                v = stg_t[pl.ds(e, _L)]
                s = stg_s[pl.ds(e, _L)]
                t = v[0]
                m = (t >= sublo) & (t < sublo + rows_w)
                tv[pl.ds(off, _L)] = v
                sv[pl.ds(off, _L)] = s
                return off + m.astype(jnp.int32)
            cnt2 = lax.fori_loop(0, cnt, _sscan, 0)
            nch = jnp.maximum((cnt2 + _CH - 1) // _CH, 1)

            def _bld(gi, si, c):
                for g in range(_CH // _L):
                    gi[pl.ds(g * _L, _L)] = tv[pl.ds(c * _CH + g * _L, _L)] + bbase
                    si[pl.ds(g * _L, _L)] = sv[pl.ds(c * _CH + g * _L, _L)] + bbase

            for h in range(2):
                ph = p0 if h == 0 else p1
                qh = q0 if h == 0 else q1

                def _zacc(r, _):
                    for c in range(_H // _L):
                        acc[pl.ds(r * _H + c * _L, _L)] = zf
                    if h == 0:
                        dacc[pl.ds(r * _L, _L)] = zf
                    return 0
                lax.fori_loop(0, rows_w, _zacc, 0)

                def _proc(pb, qb, k):
                    base = k * _CH
                    nrem = jnp.minimum(cnt2 - base, _CH)

                    def _edge(e, _2):
                        rv = tv[pl.ds(base + e, _L)]
                        ro = (rv[0] - sublo) * _H
                        for c in range(_H // _L):
                            sl = pl.ds(c * _L, _L)
                            t = pb[e, sl] + qb[e, sl]
                            ao = pl.ds(ro + c * _L, _L)
                            acc[ao] = acc[ao] + t / (1.0 + jnp.exp(-t))
                        if h == 0:
                            do = pl.ds((rv[0] - sublo) * _L, _L)
                            dacc[do] = dacc[do] + 1.0
                        return 0
                    lax.fori_loop(0, nrem, _edge, 0)

                _bld(gi0, si0, 0)
                pltpu.async_copy(ph.at[gi0], pb0, semp0)
                pltpu.async_copy(qh.at[si0], qb0, semq0)

                def _pair(k2, _):
                    c1 = jnp.minimum(2 * k2 + 1, nch - 1)
                    _bld(gi1, si1, c1)
                    pltpu.async_copy(ph.at[gi1], pb1, semp1)
                    pltpu.async_copy(qh.at[si1], qb1, semq1)
                    pltpu.make_async_copy(ph.at[gi0], pb0, semp0).wait()
                    pltpu.make_async_copy(qh.at[si0], qb0, semq0).wait()
                    _proc(pb0, qb0, 2 * k2)
                    c2 = jnp.minimum(2 * k2 + 2, nch - 1)
                    _bld(gi0, si0, c2)
                    pltpu.async_copy(ph.at[gi0], pb0, semp0)
                    pltpu.async_copy(qh.at[si0], qb0, semq0)
                    pltpu.make_async_copy(ph.at[gi1], pb1, semp1).wait()
                    pltpu.make_async_copy(qh.at[si1], qb1, semq1).wait()
                    _proc(pb1, qb1, 2 * k2 + 1)
                    return 0
                lax.fori_loop(0, (nch + 1) // 2, _pair, 0)
                pltpu.make_async_copy(ph.at[gi0], pb0, semp0).wait()
                pltpu.make_async_copy(qh.at[si0], qb0, semq0).wait()

                row0 = bbase + sublo
                s_out = s0_out if h == 0 else s1_out
                pltpu.sync_copy(acc, s_out.at[pl.ds(row0 * _H, rows_w * _H)])
                if h == 0:
                    pltpu.sync_copy(
                        dacc, deg_out.at[pl.ds(row0 * _L, rows_w * _L)])

    mesh = plsc.VectorSubcoreMesh(core_axis_name="c", subcore_axis_name="s",
                                  num_cores=_NC, num_subcores=_NS)
    return pl.kernel(
        body,
        out_type=(
            jax.ShapeDtypeStruct((BN * _H,), jnp.float32),
            jax.ShapeDtypeStruct((BN * _H,), jnp.float32),
            jax.ShapeDtypeStruct((BN * _L,), jnp.float32),
        ),
        mesh=mesh,
        scratch_types=[
            pltpu.VMEM((E + 6 * _L,), jnp.int32),
            pltpu.VMEM((E + 6 * _L,), jnp.int32),
            pltpu.VMEM((E + 6 * _L,), jnp.int32),
            pltpu.VMEM((E + 6 * _L,), jnp.int32),
            pltpu.VMEM((_CH,), jnp.int32),
            pltpu.VMEM((_CH,), jnp.int32),
            pltpu.VMEM((_CH,), jnp.int32),
            pltpu.VMEM((_CH,), jnp.int32),
            pltpu.VMEM((_CH, _H), jnp.float32),
            pltpu.VMEM((_CH, _H), jnp.float32),
            pltpu.VMEM((_CH, _H), jnp.float32),
            pltpu.VMEM((_CH, _H), jnp.float32),
            pltpu.VMEM((N1 // 8 * _H,), jnp.float32),
            pltpu.VMEM((N1 // 8 * _L,), jnp.float32),
            pltpu.SemaphoreType.DMA,
            pltpu.SemaphoreType.DMA,
            pltpu.SemaphoreType.DMA,
            pltpu.SemaphoreType.DMA,
        ],
    )


def _post_body(h_ref, s0_ref, s1_ref, deg_ref, w2m0_ref, w2m1_ref, b2m_ref,
               w1ut_ref, w1us_ref, b1u_ref, w2u_ref, b2u_ref, g_ref, bt_ref,
               o_ref):
    hb = h_ref[...]
    agg = (
        jnp.dot(s0_ref[...], w2m0_ref[...], preferred_element_type=jnp.float32)
        + jnp.dot(s1_ref[...], w2m1_ref[...], preferred_element_type=jnp.float32)
        + deg_ref[:, 0:1] * b2m_ref[...]
    )
    t = (
        jnp.dot(hb, w1ut_ref[...], preferred_element_type=jnp.float32)
        + jnp.dot(agg, w1us_ref[...], preferred_element_type=jnp.float32)
        + b1u_ref[...]
    )
    u = t * jax.nn.sigmoid(t)
    hn = jnp.dot(u, w2u_ref[...], preferred_element_type=jnp.float32) + b2u_ref[...]
    x = hb + hn
    mu = jnp.mean(x, axis=-1, keepdims=True)
    xc = x - mu
    var = jnp.mean(xc * xc, axis=-1, keepdims=True)
    o_ref[...] = xc * lax.rsqrt(var + 1e-5) * g_ref[...] + bt_ref[...]


def _post_call(hf, S0, S1, deg, w2m0, w2m1, b2m, w1ut, w1us, b1u, w2u, b2u,
               g, bt):
    BN, D = hf.shape
    H = D // 2
    BLK = 256
    full = lambda i: (0, 0)
    return pl.pallas_call(
        _post_body,
        grid=(BN // BLK,),
        in_specs=[
            pl.BlockSpec((BLK, D), lambda i: (i, 0)),
            pl.BlockSpec((BLK, H), lambda i: (i, 0)),
            pl.BlockSpec((BLK, H), lambda i: (i, 0)),
            pl.BlockSpec((BLK, _L), lambda i: (i, 0)),
            pl.BlockSpec((H, D), full),
            pl.BlockSpec((H, D), full),
            pl.BlockSpec((1, D), full),
            pl.BlockSpec((D, D), full),
            pl.BlockSpec((D, D), full),
            pl.BlockSpec((1, D), full),
            pl.BlockSpec((D, D), full),
            pl.BlockSpec((1, D), full),
            pl.BlockSpec((1, D), full),
            pl.BlockSpec((1, D), full),
        ],
        out_specs=pl.BlockSpec((BLK, D), lambda i: (i, 0)),
        out_shape=jax.ShapeDtypeStruct((BN, D), jnp.float32),
    )(hf, S0, S1, deg, w2m0, w2m1, b2m, w1ut, w1us, b1u, w2u, b2u, g, bt)


def kernel(h, edge_index, W1m, b1m, W2m, b2m, W1u, b1u, W2u, b2u, gamma, beta):
    B, N1, D = h.shape
    E = edge_index.shape[1]
    hf = h.reshape(B * N1, D)

    P0, P1, Q0, Q1 = _pre_call(hf, W1m[:D], W1m[D:], b1m.reshape(1, D))

    src = edge_index[:, :, 0]
    tgt = edge_index[:, :, 1]

    S0, S1, deg = _make_sc(B, N1, E)(tgt, src, P0, P1, Q0, Q1)
    BN = B * N1
    S0 = S0.reshape(BN, D // 2)
    S1 = S1.reshape(BN, D // 2)
    deg = deg.reshape(BN, _L)

    out = _post_call(hf, S0, S1, deg, W2m[:D // 2], W2m[D // 2:],
                     b2m.reshape(1, D), W1u[:D], W1u[D:],
                     b1u.reshape(1, D), W2u, b2u.reshape(1, D),
                     gamma.reshape(1, D), beta.reshape(1, D))
    return out.reshape(B, N1, D)


# speedup vs baseline: 1.4426x; 1.4426x over previous
import jax
import jax.numpy as jnp
from jax import lax
from jax.experimental import pallas as pl
from jax.experimental.pallas import tpu as pltpu
from jax.experimental.pallas import tpu_sc as plsc


def _pre_body(h_ref, w1t_ref, w1s_ref, b1_ref, p0_ref, p1_ref, q0_ref, q1_ref):
    h = h_ref[...]
    p = (
        jnp.dot(h, w1t_ref[...], preferred_element_type=jnp.float32)
        + b1_ref[...]
    )
    q = jnp.dot(h, w1s_ref[...], preferred_element_type=jnp.float32)
    p0_ref[...] = p[:, :128]
    p1_ref[...] = p[:, 128:]
    q0_ref[...] = q[:, :128]
    q1_ref[...] = q[:, 128:]


def _pre_call(hf, w1t, w1s, b1):
    BN, D = hf.shape
    BLK = 512
    H = D // 2
    half = jax.ShapeDtypeStruct((BN, H), jnp.float32)
    return pl.pallas_call(
        _pre_body,
        grid=(BN // BLK,),
        in_specs=[
            pl.BlockSpec((BLK, D), lambda i: (i, 0)),
            pl.BlockSpec((D, D), lambda i: (0, 0)),
            pl.BlockSpec((D, D), lambda i: (0, 0)),
            pl.BlockSpec((1, D), lambda i: (0, 0)),
        ],
        out_specs=[pl.BlockSpec((BLK, H), lambda i: (i, 0))] * 4,
        out_shape=[half, half, half, half],
    )(hf, w1t, w1s, b1)


_NC, _NS, _L = 2, 16, 16
_CH = 16
_C1 = 64
_H = 128


def _make_sc1(B, N1, E):
    ET = E // 4
    NCH = ET // _C1

    def body(tgt_h, src_h, p0, p1, q0, q1, tp0_o, tq0_o, tp1_o, tq1_o,
             tv, sv, gia, sia, gib, sib,
             ba0, ba1, ba2, ba3, bb0, bb1, bb2, bb3,
             sa0, sa1, sa2, sa3, sb0, sb1, sb2, sb3):
        cid = lax.axis_index("c")
        sid = lax.axis_index("s")
        wid = cid * _NS + sid
        b = wid // 4
        eoff = (wid % 4) * ET
        bbase = b * N1
        rbase = b * E + eoff

        pltpu.sync_copy(tgt_h.at[b], tv)
        pltpu.sync_copy(src_h.at[b], sv)

        sets = [
            (gia, sia, (ba0, ba1, ba2, ba3), (sa0, sa1, sa2, sa3)),
            (gib, sib, (bb0, bb1, bb2, bb3), (sb0, sb1, sb2, sb3)),
        ]
        outs = (tp0_o, tq0_o, tp1_o, tq1_o)

        def bld(st, c):
            gi, si = st[0], st[1]
            for g in range(_C1 // _L):
                sl = pl.ds(eoff + c * _C1 + g * _L, _L)
                gi[pl.ds(g * _L, _L)] = tv[sl] + bbase
                si[pl.ds(g * _L, _L)] = sv[sl] + bbase

        def srcs(st):
            gi, si = st[0], st[1]
            return (p0.at[gi], q0.at[si], p1.at[gi], q1.at[si])

        def issue_gathers(st):
            for s, d, m in zip(srcs(st), st[2], st[3]):
                pltpu.async_copy(s, d, m)

        def wait_gathers(st):
            for s, d, m in zip(srcs(st), st[2], st[3]):
                pltpu.make_async_copy(s, d, m).wait()

        def issue_writes(st, c):
            r = pl.ds(rbase + c * _C1, _C1)
            for d, buf, m in zip(outs, st[2], st[3]):
                pltpu.async_copy(buf, d.at[r], m)

        def wait_writes(st, c):
            r = pl.ds(rbase + c * _C1, _C1)
            for d, buf, m in zip(outs, st[2], st[3]):
                pltpu.make_async_copy(buf, d.at[r], m).wait()

        for c in range(NCH):
            st = sets[c % 2]
            if c >= 2:
                wait_writes(st, c - 2)
            bld(st, c)
            issue_gathers(st)
            if c >= 1:
                prev = sets[(c - 1) % 2]
                wait_gathers(prev)
                issue_writes(prev, c - 1)
        last = sets[(NCH - 1) % 2]
        wait_gathers(last)
        issue_writes(last, NCH - 1)
        wait_writes(sets[(NCH - 2) % 2], NCH - 2)
        wait_writes(last, NCH - 1)

    mesh = plsc.VectorSubcoreMesh(core_axis_name="c", subcore_axis_name="s",
                                  num_cores=_NC, num_subcores=_NS)
    edge = jax.ShapeDtypeStruct((B * E, _H), jnp.float32)
    return pl.kernel(
        body,
        out_type=(edge, edge, edge, edge),
        mesh=mesh,
        scratch_types=(
            [pltpu.VMEM((E,), jnp.int32)] * 2
            + [pltpu.VMEM((_C1,), jnp.int32)] * 4
            + [pltpu.VMEM((_C1, _H), jnp.float32)] * 8
            + [pltpu.SemaphoreType.DMA] * 8
        ),
    )


def _silu_body(tp0_ref, tq0_ref, tp1_ref, tq1_ref, u0_ref, u1_ref):
    t0 = tp0_ref[...] + tq0_ref[...]
    u0_ref[...] = t0 / (1.0 + jnp.exp(-t0))
    t1 = tp1_ref[...] + tq1_ref[...]
    u1_ref[...] = t1 / (1.0 + jnp.exp(-t1))


def _silu_call(tp0, tq0, tp1, tq1):
    BE, H = tp0.shape
    BLK = 1024
    edge = jax.ShapeDtypeStruct((BE, H), jnp.float32)
    return pl.pallas_call(
        _silu_body,
        grid=(BE // BLK,),
        in_specs=[pl.BlockSpec((BLK, H), lambda i: (i, 0))] * 4,
        out_specs=[pl.BlockSpec((BLK, H), lambda i: (i, 0))] * 2,
        out_shape=[edge, edge],
    )(tp0, tq0, tp1, tq1)


def _make_sc2(B, N1, E):
    BN = B * N1
    rows_q = N1 // 4
    rows_w = rows_q // 2
    pad = E + 6 * _L

    def body(tgt_h, iota_h, u0, u1, s0_out, s1_out, deg_out,
             tv, sv, stg_t, stg_s, iv, si0, si1,
             ub0, ub1, acc, dacc, sem0, sem1):
        cid = lax.axis_index("c")
        sid = lax.axis_index("s")
        wid = cid * _NS + sid
        b = wid // 4
        lo = (wid % 4) * rows_q
        ebase = b * E

        pltpu.sync_copy(tgt_h.at[b], tv.at[pl.ds(0, E)])
        pltpu.sync_copy(iota_h, iv)

        v0 = tv[pl.ds(0, _L)]
        zi = v0 - v0
        zf = zi.astype(jnp.float32)

        def _zpad(g, _):
            tv[pl.ds(E + g * _L, _L)] = zi
            sv[pl.ds(E + g * _L, _L)] = zi
            return 0
        lax.fori_loop(0, 6, _zpad, 0)

        def _zstg(g, _):
            stg_t[pl.ds(g * _L, _L)] = zi
            stg_s[pl.ds(g * _L, _L)] = zi
            return 0
        lax.fori_loop(0, pad // _L, _zstg, 0)

        def _scan(e, off):
            v = tv[pl.ds(e, _L)]
            p = iv[pl.ds(e, _L)]
            t = v[0]
            m = (t >= lo) & (t < lo + rows_q)
            stg_t[pl.ds(off, _L)] = v
            stg_s[pl.ds(off, _L)] = p
            return off + m.astype(jnp.int32)
        cnt = lax.fori_loop(0, E, _scan, 0)

        for sub in range(2):
            sublo = lo + sub * rows_w
            def _sscan(e, off):
                v = stg_t[pl.ds(e, _L)]
                p = stg_s[pl.ds(e, _L)]
                t = v[0]
                m = (t >= sublo) & (t < sublo + rows_w)
                tv[pl.ds(off, _L)] = v
                sv[pl.ds(off, _L)] = p
                return off + m.astype(jnp.int32)
            cnt2 = lax.fori_loop(0, cnt, _sscan, 0)
            nch = jnp.maximum((cnt2 + _CH - 1) // _CH, 1)

            def _bld(si, c):
                si[pl.ds(0, _L)] = sv[pl.ds(c * _CH, _L)] + ebase

            for h in range(2):
                uh = u0 if h == 0 else u1

                def _zacc(r, _):
                    for c in range(_H // _L):
                        acc[pl.ds(r * _H + c * _L, _L)] = zf
                    if h == 0:
                        dacc[pl.ds(r * _L, _L)] = zf
                    return 0
                lax.fori_loop(0, rows_w, _zacc, 0)

                def _proc(ub, k):
                    base = k * _CH
                    nrem = jnp.minimum(cnt2 - base, _CH)

                    def _edge(e, _2):
                        rv = tv[pl.ds(base + e, _L)]
                        ro = (rv[0] - sublo) * _H
                        for c in range(_H // _L):
                            ao = pl.ds(ro + c * _L, _L)
                            acc[ao] = acc[ao] + ub[e, pl.ds(c * _L, _L)]
                        if h == 0:
                            do = pl.ds((rv[0] - sublo) * _L, _L)
                            dacc[do] = dacc[do] + 1.0
                        return 0
                    lax.fori_loop(0, nrem, _edge, 0)

                _bld(si0, 0)
                pltpu.async_copy(uh.at[si0], ub0, sem0)

                def _pair(k2, _):
                    c1 = jnp.minimum(2 * k2 + 1, nch - 1)
                    _bld(si1, c1)
                    pltpu.async_copy(uh.at[si1], ub1, sem1)
                    pltpu.make_async_copy(uh.at[si0], ub0, sem0).wait()
                    _proc(ub0, 2 * k2)
                    c2 = jnp.minimum(2 * k2 + 2, nch - 1)
                    _bld(si0, c2)
                    pltpu.async_copy(uh.at[si0], ub0, sem0)
                    pltpu.make_async_copy(uh.at[si1], ub1, sem1).wait()
                    _proc(ub1, 2 * k2 + 1)
                    return 0
                lax.fori_loop(0, (nch + 1) // 2, _pair, 0)
                pltpu.make_async_copy(uh.at[si0], ub0, sem0).wait()

                row0 = b * N1 + sublo
                s_out = s0_out if h == 0 else s1_out
                pltpu.sync_copy(acc, s_out.at[pl.ds(row0 * _H, rows_w * _H)])
                if h == 0:
                    pltpu.sync_copy(
                        dacc, deg_out.at[pl.ds(row0 * _L, rows_w * _L)])

    mesh = plsc.VectorSubcoreMesh(core_axis_name="c", subcore_axis_name="s",
                                  num_cores=_NC, num_subcores=_NS)
    return pl.kernel(
        body,
        out_type=(
            jax.ShapeDtypeStruct((BN * _H,), jnp.float32),
            jax.ShapeDtypeStruct((BN * _H,), jnp.float32),
            jax.ShapeDtypeStruct((BN * _L,), jnp.float32),
        ),
        mesh=mesh,
        scratch_types=[
            pltpu.VMEM((E + 6 * _L,), jnp.int32),
            pltpu.VMEM((E + 6 * _L,), jnp.int32),
            pltpu.VMEM((E + 6 * _L,), jnp.int32),
            pltpu.VMEM((E + 6 * _L,), jnp.int32),
            pltpu.VMEM((E + 6 * _L,), jnp.int32),
            pltpu.VMEM((_CH,), jnp.int32),
            pltpu.VMEM((_CH,), jnp.int32),
            pltpu.VMEM((_CH, _H), jnp.float32),
            pltpu.VMEM((_CH, _H), jnp.float32),
            pltpu.VMEM((N1 // 8 * _H,), jnp.float32),
            pltpu.VMEM((N1 // 8 * _L,), jnp.float32),
            pltpu.SemaphoreType.DMA,
            pltpu.SemaphoreType.DMA,
        ],
    )


def _post_body(h_ref, s0_ref, s1_ref, deg_ref, w2m0_ref, w2m1_ref, b2m_ref,
               w1ut_ref, w1us_ref, b1u_ref, w2u_ref, b2u_ref, g_ref, bt_ref,
               o_ref):
    hb = h_ref[...]
    agg = (
        jnp.dot(s0_ref[...], w2m0_ref[...], preferred_element_type=jnp.float32)
        + jnp.dot(s1_ref[...], w2m1_ref[...], preferred_element_type=jnp.float32)
        + deg_ref[:, 0:1] * b2m_ref[...]
    )
    t = (
        jnp.dot(hb, w1ut_ref[...], preferred_element_type=jnp.float32)
        + jnp.dot(agg, w1us_ref[...], preferred_element_type=jnp.float32)
        + b1u_ref[...]
    )
    u = t * jax.nn.sigmoid(t)
    hn = jnp.dot(u, w2u_ref[...], preferred_element_type=jnp.float32) + b2u_ref[...]
    x = hb + hn
    mu = jnp.mean(x, axis=-1, keepdims=True)
    xc = x - mu
    var = jnp.mean(xc * xc, axis=-1, keepdims=True)
    o_ref[...] = xc * lax.rsqrt(var + 1e-5) * g_ref[...] + bt_ref[...]


def _post_call(hf, S0, S1, deg, w2m0, w2m1, b2m, w1ut, w1us, b1u, w2u, b2u,
               g, bt):
    BN, D = hf.shape
    H = D // 2
    BLK = 256
    full = lambda i: (0, 0)
    return pl.pallas_call(
        _post_body,
        grid=(BN // BLK,),
        in_specs=[
            pl.BlockSpec((BLK, D), lambda i: (i, 0)),
            pl.BlockSpec((BLK, H), lambda i: (i, 0)),
            pl.BlockSpec((BLK, H), lambda i: (i, 0)),
            pl.BlockSpec((BLK, _L), lambda i: (i, 0)),
            pl.BlockSpec((H, D), full),
            pl.BlockSpec((H, D), full),
            pl.BlockSpec((1, D), full),
            pl.BlockSpec((D, D), full),
            pl.BlockSpec((D, D), full),
            pl.BlockSpec((1, D), full),
            pl.BlockSpec((D, D), full),
            pl.BlockSpec((1, D), full),
            pl.BlockSpec((1, D), full),
            pl.BlockSpec((1, D), full),
        ],
        out_specs=pl.BlockSpec((BLK, D), lambda i: (i, 0)),
        out_shape=jax.ShapeDtypeStruct((BN, D), jnp.float32),
    )(hf, S0, S1, deg, w2m0, w2m1, b2m, w1ut, w1us, b1u, w2u, b2u, g, bt)


def kernel(h, edge_index, W1m, b1m, W2m, b2m, W1u, b1u, W2u, b2u, gamma, beta):
    B, N1, D = h.shape
    E = edge_index.shape[1]
    hf = h.reshape(B * N1, D)

    P0, P1, Q0, Q1 = _pre_call(hf, W1m[:D], W1m[D:], b1m.reshape(1, D))

    src = edge_index[:, :, 0]
    tgt = edge_index[:, :, 1]
    iota = jnp.arange(E + 6 * _L, dtype=jnp.int32)

    TP0, TQ0, TP1, TQ1 = _make_sc1(B, N1, E)(tgt, src, P0, P1, Q0, Q1)
    U0, U1 = _silu_call(TP0, TQ0, TP1, TQ1)
    S0, S1, deg = _make_sc2(B, N1, E)(tgt, iota, U0, U1)

    BN = B * N1
    S0 = S0.reshape(BN, D // 2)
    S1 = S1.reshape(BN, D // 2)
    deg = deg.reshape(BN, _L)

    out = _post_call(hf, S0, S1, deg, W2m[:D // 2], W2m[D // 2:],
                     b2m.reshape(1, D), W1u[:D], W1u[D:],
                     b1u.reshape(1, D), W2u, b2u.reshape(1, D),
                     gamma.reshape(1, D), beta.reshape(1, D))
    return out.reshape(B, N1, D)


# speedup vs baseline: 1.4791x; 1.0253x over previous
import jax
import jax.numpy as jnp
from jax import lax
from jax.experimental import pallas as pl
from jax.experimental.pallas import tpu as pltpu
from jax.experimental.pallas import tpu_sc as plsc


def _pre_body(h_ref, w1t_ref, w1s_ref, b1_ref, p0_ref, p1_ref, q0_ref, q1_ref):
    h = h_ref[...]
    p = (
        jnp.dot(h, w1t_ref[...], preferred_element_type=jnp.float32)
        + b1_ref[...]
    )
    q = jnp.dot(h, w1s_ref[...], preferred_element_type=jnp.float32)
    p0_ref[...] = p[:, :128]
    p1_ref[...] = p[:, 128:]
    q0_ref[...] = q[:, :128]
    q1_ref[...] = q[:, 128:]


def _pre_call(hf, w1t, w1s, b1):
    BN, D = hf.shape
    BLK = 512
    H = D // 2
    half = jax.ShapeDtypeStruct((BN, H), jnp.float32)
    return pl.pallas_call(
        _pre_body,
        grid=(BN // BLK,),
        in_specs=[
            pl.BlockSpec((BLK, D), lambda i: (i, 0)),
            pl.BlockSpec((D, D), lambda i: (0, 0)),
            pl.BlockSpec((D, D), lambda i: (0, 0)),
            pl.BlockSpec((1, D), lambda i: (0, 0)),
        ],
        out_specs=[pl.BlockSpec((BLK, H), lambda i: (i, 0))] * 4,
        out_shape=[half, half, half, half],
    )(hf, w1t, w1s, b1)


_NC, _NS, _L = 2, 16, 16
_CH = 16
_C1 = 64
_H = 128


def _make_sc1(B, N1, E):
    ET = E // 4
    NCH = ET // _C1

    def body(tgt_h, src_h, p0, p1, q0, q1, tp0_o, tq0_o, tp1_o, tq1_o,
             tv, sv, gia, sia, gib, sib,
             ba0, ba1, ba2, ba3, bb0, bb1, bb2, bb3,
             sa0, sa1, sa2, sa3, sb0, sb1, sb2, sb3):
        cid = lax.axis_index("c")
        sid = lax.axis_index("s")
        wid = cid * _NS + sid
        b = wid // 4
        eoff = (wid % 4) * ET
        bbase = b * N1
        rbase = b * E + eoff

        pltpu.sync_copy(tgt_h.at[b], tv)
        pltpu.sync_copy(src_h.at[b], sv)

        sets = [
            (gia, sia, (ba0, ba1, ba2, ba3), (sa0, sa1, sa2, sa3)),
            (gib, sib, (bb0, bb1, bb2, bb3), (sb0, sb1, sb2, sb3)),
        ]
        outs = (tp0_o, tq0_o, tp1_o, tq1_o)

        def bld(st, c):
            gi, si = st[0], st[1]
            for g in range(_C1 // _L):
                sl = pl.ds(eoff + c * _C1 + g * _L, _L)
                gi[pl.ds(g * _L, _L)] = tv[sl] + bbase
                si[pl.ds(g * _L, _L)] = sv[sl] + bbase

        def srcs(st):
            gi, si = st[0], st[1]
            return (p0.at[gi], q0.at[si], p1.at[gi], q1.at[si])

        def issue_gathers(st):
            for s, d, m in zip(srcs(st), st[2], st[3]):
                pltpu.async_copy(s, d, m)

        def wait_gathers(st):
            for s, d, m in zip(srcs(st), st[2], st[3]):
                pltpu.make_async_copy(s, d, m).wait()

        def issue_writes(st, c):
            r = pl.ds(rbase + c * _C1, _C1)
            for d, buf, m in zip(outs, st[2], st[3]):
                pltpu.async_copy(buf, d.at[r], m)

        def wait_writes(st, c):
            r = pl.ds(rbase + c * _C1, _C1)
            for d, buf, m in zip(outs, st[2], st[3]):
                pltpu.make_async_copy(buf, d.at[r], m).wait()

        for c in range(NCH):
            st = sets[c % 2]
            if c >= 2:
                wait_writes(st, c - 2)
            bld(st, c)
            issue_gathers(st)
            if c >= 1:
                prev = sets[(c - 1) % 2]
                wait_gathers(prev)
                issue_writes(prev, c - 1)
        last = sets[(NCH - 1) % 2]
        wait_gathers(last)
        issue_writes(last, NCH - 1)
        wait_writes(sets[(NCH - 2) % 2], NCH - 2)
        wait_writes(last, NCH - 1)

    mesh = plsc.VectorSubcoreMesh(core_axis_name="c", subcore_axis_name="s",
                                  num_cores=_NC, num_subcores=_NS)
    edge = jax.ShapeDtypeStruct((B * E, _H), jnp.float32)
    return pl.kernel(
        body,
        out_type=(edge, edge, edge, edge),
        mesh=mesh,
        scratch_types=(
            [pltpu.VMEM((E,), jnp.int32)] * 2
            + [pltpu.VMEM((_C1,), jnp.int32)] * 4
            + [pltpu.VMEM((_C1, _H), jnp.float32)] * 8
            + [pltpu.SemaphoreType.DMA] * 8
        ),
    )


def _silu_body(tp0_ref, tq0_ref, tp1_ref, tq1_ref, u0_ref, u1_ref):
    t0 = tp0_ref[...] + tq0_ref[...]
    u0_ref[...] = t0 / (1.0 + jnp.exp(-t0))
    t1 = tp1_ref[...] + tq1_ref[...]
    u1_ref[...] = t1 / (1.0 + jnp.exp(-t1))


def _silu_call(tp0, tq0, tp1, tq1):
    BE, H = tp0.shape
    BLK = 1024
    edge = jax.ShapeDtypeStruct((BE, H), jnp.float32)
    return pl.pallas_call(
        _silu_body,
        grid=(BE // BLK,),
        in_specs=[pl.BlockSpec((BLK, H), lambda i: (i, 0))] * 4,
        out_specs=[pl.BlockSpec((BLK, H), lambda i: (i, 0))] * 2,
        out_shape=[edge, edge],
    )(tp0, tq0, tp1, tq1)


def _make_sc2(B, N1, E):
    BN = B * N1
    rows_q = N1 // 4
    rows_w = rows_q // 2
    pad = E + 6 * _L

    def body(tgt_h, iota_h, u0, u1, s0_out, s1_out, deg_out,
             tv, sv, stg_t, stg_s, iv, si0, si1,
             ub0, ub1, acc, dacc, sem0, sem1):
        cid = lax.axis_index("c")
        sid = lax.axis_index("s")
        wid = cid * _NS + sid
        b = wid // 4
        lo = (wid % 4) * rows_q
        ebase = b * E

        pltpu.sync_copy(tgt_h.at[b], tv.at[pl.ds(0, E)])
        pltpu.sync_copy(iota_h, iv)

        v0 = tv[pl.ds(0, _L)]
        zi = v0 - v0
        zf = zi.astype(jnp.float32)

        def _zpad(g, _):
            tv[pl.ds(E + g * _L, _L)] = zi
            sv[pl.ds(E + g * _L, _L)] = zi
            return 0
        lax.fori_loop(0, 6, _zpad, 0)

        def _zstg(g, _):
            stg_t[pl.ds(g * _L, _L)] = zi
            stg_s[pl.ds(g * _L, _L)] = zi
            return 0
        lax.fori_loop(0, pad // _L, _zstg, 0)

        def _scan4(i, off):
            for j in range(4):
                e = i * 4 + j
                v = tv[pl.ds(e, _L)]
                p = iv[pl.ds(e, _L)]
                t = v[0]
                m = (t >= lo) & (t < lo + rows_q)
                stg_t[pl.ds(off, _L)] = v
                stg_s[pl.ds(off, _L)] = p
                off = off + m.astype(jnp.int32)
            return off
        cnt = lax.fori_loop(0, E // 4, _scan4, 0)

        for sub in range(2):
            sublo = lo + sub * rows_w
            def _sstep(e, off):
                v = stg_t[pl.ds(e, _L)]
                p = stg_s[pl.ds(e, _L)]
                t = v[0]
                m = (t >= sublo) & (t < sublo + rows_w)
                tv[pl.ds(off, _L)] = v
                sv[pl.ds(off, _L)] = p
                return off + m.astype(jnp.int32)

            def _sscan4(i, off):
                for j in range(4):
                    off = _sstep(i * 4 + j, off)
                return off
            cnt2 = lax.fori_loop(0, cnt // 4, _sscan4, 0)
            cnt2 = lax.fori_loop(4 * (cnt // 4), cnt, _sstep, cnt2)
            nch = jnp.maximum((cnt2 + _CH - 1) // _CH, 1)

            def _bld(si, c):
                si[pl.ds(0, _L)] = sv[pl.ds(c * _CH, _L)] + ebase

            for h in range(2):
                uh = u0 if h == 0 else u1

                def _zacc(r, _):
                    for c in range(_H // _L):
                        acc[pl.ds(r * _H + c * _L, _L)] = zf
                    if h == 0:
                        dacc[pl.ds(r * _L, _L)] = zf
                    return 0
                lax.fori_loop(0, rows_w, _zacc, 0)

                def _proc(ub, k):
                    base = k * _CH
                    nrem = jnp.minimum(cnt2 - base, _CH)

                    def _edge(e, _2):
                        rv = tv[pl.ds(base + e, _L)]
                        ro = (rv[0] - sublo) * _H
                        for c in range(_H // _L):
                            ao = pl.ds(ro + c * _L, _L)
                            acc[ao] = acc[ao] + ub[e, pl.ds(c * _L, _L)]
                        if h == 0:
                            do = pl.ds((rv[0] - sublo) * _L, _L)
                            dacc[do] = dacc[do] + 1.0
                        return 0
                    lax.fori_loop(0, nrem, _edge, 0)

                _bld(si0, 0)
                pltpu.async_copy(uh.at[si0], ub0, sem0)

                def _pair(k2, _):
                    c1 = jnp.minimum(2 * k2 + 1, nch - 1)
                    _bld(si1, c1)
                    pltpu.async_copy(uh.at[si1], ub1, sem1)
                    pltpu.make_async_copy(uh.at[si0], ub0, sem0).wait()
                    _proc(ub0, 2 * k2)
                    c2 = jnp.minimum(2 * k2 + 2, nch - 1)
                    _bld(si0, c2)
                    pltpu.async_copy(uh.at[si0], ub0, sem0)
                    pltpu.make_async_copy(uh.at[si1], ub1, sem1).wait()
                    _proc(ub1, 2 * k2 + 1)
                    return 0
                lax.fori_loop(0, (nch + 1) // 2, _pair, 0)
                pltpu.make_async_copy(uh.at[si0], ub0, sem0).wait()

                row0 = b * N1 + sublo
                s_out = s0_out if h == 0 else s1_out
                pltpu.sync_copy(acc, s_out.at[pl.ds(row0 * _H, rows_w * _H)])
                if h == 0:
                    pltpu.sync_copy(
                        dacc, deg_out.at[pl.ds(row0 * _L, rows_w * _L)])

    mesh = plsc.VectorSubcoreMesh(core_axis_name="c", subcore_axis_name="s",
                                  num_cores=_NC, num_subcores=_NS)
    return pl.kernel(
        body,
        out_type=(
            jax.ShapeDtypeStruct((BN * _H,), jnp.float32),
            jax.ShapeDtypeStruct((BN * _H,), jnp.float32),
            jax.ShapeDtypeStruct((BN * _L,), jnp.float32),
        ),
        mesh=mesh,
        scratch_types=[
            pltpu.VMEM((E + 6 * _L,), jnp.int32),
            pltpu.VMEM((E + 6 * _L,), jnp.int32),
            pltpu.VMEM((E + 6 * _L,), jnp.int32),
            pltpu.VMEM((E + 6 * _L,), jnp.int32),
            pltpu.VMEM((E + 6 * _L,), jnp.int32),
            pltpu.VMEM((_CH,), jnp.int32),
            pltpu.VMEM((_CH,), jnp.int32),
            pltpu.VMEM((_CH, _H), jnp.float32),
            pltpu.VMEM((_CH, _H), jnp.float32),
            pltpu.VMEM((N1 // 8 * _H,), jnp.float32),
            pltpu.VMEM((N1 // 8 * _L,), jnp.float32),
            pltpu.SemaphoreType.DMA,
            pltpu.SemaphoreType.DMA,
        ],
    )


def _post_body(h_ref, s0_ref, s1_ref, deg_ref, w2m0_ref, w2m1_ref, b2m_ref,
               w1ut_ref, w1us_ref, b1u_ref, w2u_ref, b2u_ref, g_ref, bt_ref,
               o_ref):
    hb = h_ref[...]
    agg = (
        jnp.dot(s0_ref[...], w2m0_ref[...], preferred_element_type=jnp.float32)
        + jnp.dot(s1_ref[...], w2m1_ref[...], preferred_element_type=jnp.float32)
        + deg_ref[:, 0:1] * b2m_ref[...]
    )
    t = (
        jnp.dot(hb, w1ut_ref[...], preferred_element_type=jnp.float32)
        + jnp.dot(agg, w1us_ref[...], preferred_element_type=jnp.float32)
        + b1u_ref[...]
    )
    u = t * jax.nn.sigmoid(t)
    hn = jnp.dot(u, w2u_ref[...], preferred_element_type=jnp.float32) + b2u_ref[...]
    x = hb + hn
    mu = jnp.mean(x, axis=-1, keepdims=True)
    xc = x - mu
    var = jnp.mean(xc * xc, axis=-1, keepdims=True)
    o_ref[...] = xc * lax.rsqrt(var + 1e-5) * g_ref[...] + bt_ref[...]


def _post_call(hf, S0, S1, deg, w2m0, w2m1, b2m, w1ut, w1us, b1u, w2u, b2u,
               g, bt):
    BN, D = hf.shape
    H = D // 2
    BLK = 256
    full = lambda i: (0, 0)
    return pl.pallas_call(
        _post_body,
        grid=(BN // BLK,),
        in_specs=[
            pl.BlockSpec((BLK, D), lambda i: (i, 0)),
            pl.BlockSpec((BLK, H), lambda i: (i, 0)),
            pl.BlockSpec((BLK, H), lambda i: (i, 0)),
            pl.BlockSpec((BLK, _L), lambda i: (i, 0)),
            pl.BlockSpec((H, D), full),
            pl.BlockSpec((H, D), full),
            pl.BlockSpec((1, D), full),
            pl.BlockSpec((D, D), full),
            pl.BlockSpec((D, D), full),
            pl.BlockSpec((1, D), full),
            pl.BlockSpec((D, D), full),
            pl.BlockSpec((1, D), full),
            pl.BlockSpec((1, D), full),
            pl.BlockSpec((1, D), full),
        ],
        out_specs=pl.BlockSpec((BLK, D), lambda i: (i, 0)),
        out_shape=jax.ShapeDtypeStruct((BN, D), jnp.float32),
    )(hf, S0, S1, deg, w2m0, w2m1, b2m, w1ut, w1us, b1u, w2u, b2u, g, bt)


def kernel(h, edge_index, W1m, b1m, W2m, b2m, W1u, b1u, W2u, b2u, gamma, beta):
    B, N1, D = h.shape
    E = edge_index.shape[1]
    hf = h.reshape(B * N1, D)

    P0, P1, Q0, Q1 = _pre_call(hf, W1m[:D], W1m[D:], b1m.reshape(1, D))

    src = edge_index[:, :, 0]
    tgt = edge_index[:, :, 1]
    iota = jnp.arange(E + 6 * _L, dtype=jnp.int32)

    TP0, TQ0, TP1, TQ1 = _make_sc1(B, N1, E)(tgt, src, P0, P1, Q0, Q1)
    U0, U1 = _silu_call(TP0, TQ0, TP1, TQ1)
    S0, S1, deg = _make_sc2(B, N1, E)(tgt, iota, U0, U1)

    BN = B * N1
    S0 = S0.reshape(BN, D // 2)
    S1 = S1.reshape(BN, D // 2)
    deg = deg.reshape(BN, _L)

    out = _post_call(hf, S0, S1, deg, W2m[:D // 2], W2m[D // 2:],
                     b2m.reshape(1, D), W1u[:D], W1u[D:],
                     b1u.reshape(1, D), W2u, b2u.reshape(1, D),
                     gamma.reshape(1, D), beta.reshape(1, D))
    return out.reshape(B, N1, D)
